# Initial kernel scaffold; baseline (speedup 1.0000x reference)
import jax
import jax.numpy as jnp
from jax.experimental import pallas as pl


def kernel(x, edge_index, edge_attr, batch, W1, b1, p1, W2, b2, p2, Wf1, bf1, g1, be1, Wf2, bf2, g2, be2, Wf3, bf3, Ws, bs):
    raise NotImplementedError("write your pallas kernel here")


# speedup vs baseline: 20.6543x; 20.6543x over previous
import functools
import math

import jax
import jax.numpy as jnp
from jax import lax
from jax.experimental import pallas as pl
from jax.experimental.pallas import tpu as pltpu
from jax.experimental.pallas import tpu_sc as plsc

N = 50000
E = 800000
F_IN = 128
DIM = 64
K1 = 25000
K2 = 12500
P1 = 25600
P2 = 12800
M2 = 25600
NP = 50176
ECH1 = 400
ESUB = E // 16

_mesh = plsc.VectorSubcoreMesh(core_axis_name="c", subcore_axis_name="s")
_sc_params = pltpu.CompilerParams(use_tc_tiling_on_sc=False)


def _mm_body(x_ref, w_ref, o_ref):
    o_ref[...] = jnp.dot(x_ref[...], w_ref[...],
                         preferred_element_type=jnp.float32)


def _mm(x, w, bm=400):
    m = x.shape[0]
    k = x.shape[1]
    nb = m // bm
    return pl.pallas_call(
        _mm_body,
        grid=(nb,),
        in_specs=[
            pl.BlockSpec((bm, k), lambda i: (i, 0)),
            pl.BlockSpec((k, DIM), lambda i: (0, 0)),
        ],
        out_specs=pl.BlockSpec((bm, DIM), lambda i: (i, 0)),
        out_shape=jax.ShapeDtypeStruct((m, DIM), jnp.float32),
    )(x, w)


def _bias_score_body(sa_ref, sb_ref, b_ref, p_ref, h_ref, sc_ref):
    hh = jnp.concatenate([sa_ref[...], sb_ref[...]], axis=1) + b_ref[...]
    hh = jnp.maximum(hh, 0.0)
    h_ref[...] = hh
    pv = p_ref[...]
    inv = lax.rsqrt(jnp.sum(pv * pv))
    sv = jnp.dot(hh, pv.reshape(DIM, 1), preferred_element_type=jnp.float32)
    sc_ref[...] = jnp.tanh(sv * inv)


def _bias_score(sa, sb, b, p, bm=1024):
    m = sa.shape[0]
    nb = m // bm
    return pl.pallas_call(
        _bias_score_body,
        grid=(nb,),
        in_specs=[
            pl.BlockSpec((bm, 32), lambda i: (i, 0)),
            pl.BlockSpec((bm, 32), lambda i: (i, 0)),
            pl.BlockSpec((1, DIM), lambda i: (0, 0)),
            pl.BlockSpec((1, DIM), lambda i: (0, 0)),
        ],
        out_specs=[
            pl.BlockSpec((bm, DIM), lambda i: (i, 0)),
            pl.BlockSpec((bm, 1), lambda i: (i, 0)),
        ],
        out_shape=[
            jax.ShapeDtypeStruct((m, DIM), jnp.float32),
            jax.ShapeDtypeStruct((m, 1), jnp.float32),
        ],
    )(sa, sb, b.reshape(1, DIM), p.reshape(1, DIM))


def _seg1_body(src_hbm, dst_hbm, hcat_hbm, z_hbm, o_hbm,
               eidx_v, didx_v, rows_v, acc_sh, sem):
    c = lax.axis_index("c")
    s = lax.axis_index("s")
    pltpu.sync_copy(z_hbm.at[pl.ds(0, 3136)], acc_sh.at[pl.ds(s * 3136, 3136)])
    plsc.subcore_barrier()
    coff = c * N

    def chunk(j, carry):
        off = s * ESUB + j * ECH1
        pltpu.sync_copy(src_hbm.at[pl.ds(off, ECH1)], eidx_v)
        pltpu.sync_copy(dst_hbm.at[pl.ds(off, ECH1)], didx_v)

        def adj(g, cc):
            eidx_v[pl.ds(g * 16, 16)] = eidx_v[pl.ds(g * 16, 16)] + coff
            return cc

        lax.fori_loop(0, ECH1 // 16, adj, 0)
        pltpu.async_copy(hcat_hbm.at[eidx_v], rows_v, sem).wait()
        pltpu.sync_copy(rows_v, acc_sh.at[didx_v], add=True)
        return carry

    lax.fori_loop(0, ESUB // ECH1, chunk, 0)
    plsc.subcore_barrier()
    r0 = s * 3136
    pltpu.sync_copy(acc_sh.at[pl.ds(r0, 3136)],
                    o_hbm.at[pl.ds(c * NP + r0, 3136)])


_seg1 = pl.kernel(
    _seg1_body,
    out_type=jax.ShapeDtypeStruct((2 * NP, 32), jnp.float32),
    mesh=_mesh,
    compiler_params=_sc_params,
    scratch_types=[
        pltpu.VMEM((ECH1,), jnp.int32),
        pltpu.VMEM((ECH1,), jnp.int32),
        pltpu.VMEM((ECH1, 32), jnp.float32),
        pltpu.VMEM_SHARED((NP, 32), jnp.float32),
        pltpu.SemaphoreType.DMA,
    ],
)


def _gather_body(pw, perm_hbm, h_hbm, xg_hbm, pidx_v, rows_v, sem):
    c = lax.axis_index("c")
    s = lax.axis_index("s")
    w = s * 2 + c
    base = w * pw
    pltpu.sync_copy(perm_hbm.at[pl.ds(base, pw)], pidx_v)
    pltpu.async_copy(h_hbm.at[pidx_v], rows_v, sem).wait()
    pltpu.sync_copy(rows_v, xg_hbm.at[pl.ds(base, pw)])


def _make_gather(total, pw):
    return pl.kernel(
        functools.partial(_gather_body, pw),
        out_type=jax.ShapeDtypeStruct((total, DIM), jnp.float32),
        mesh=_mesh,
        compiler_params=_sc_params,
        scratch_types=[
            pltpu.VMEM((pw,), jnp.int32),
            pltpu.VMEM((pw, DIM), jnp.float32),
            pltpu.SemaphoreType.DMA,
        ],
    )


_gather1 = _make_gather(P1, P1 // 32)
_gather2 = _make_gather(P2, P2 // 32)


def _rowscatter_body(perm_hbm, hc_hbm, z_hbm, o_hbm,
                     pidx_v, rows_v, acc_sh, sem):
    c = lax.axis_index("c")
    s = lax.axis_index("s")
    pltpu.sync_copy(z_hbm.at[pl.ds(0, 3136)], acc_sh.at[pl.ds(s * 3136, 3136)])
    plsc.subcore_barrier()

    def chunk(j, carry):
        off = s * (P1 // 16) + j * 400
        pltpu.sync_copy(perm_hbm.at[pl.ds(off, 400)], pidx_v)
        pltpu.sync_copy(hc_hbm.at[pl.ds(c * P1 + off, 400)], rows_v)
        pltpu.sync_copy(rows_v, acc_sh.at[pidx_v], add=True)
        return carry

    lax.fori_loop(0, P1 // 16 // 400, chunk, 0)
    plsc.subcore_barrier()
    r0 = s * 3136
    pltpu.sync_copy(acc_sh.at[pl.ds(r0, 3136)],
                    o_hbm.at[pl.ds(c * NP + r0, 3136)])


_rowscatter = pl.kernel(
    _rowscatter_body,
    out_type=jax.ShapeDtypeStruct((2 * NP, 32), jnp.float32),
    mesh=_mesh,
    compiler_params=_sc_params,
    scratch_types=[
        pltpu.VMEM((400,), jnp.int32),
        pltpu.VMEM((400, 32), jnp.float32),
        pltpu.VMEM_SHARED((NP, 32), jnp.float32),
        pltpu.SemaphoreType.DMA,
    ],
)


def _gatherh_body(perm_hbm, o2_hbm, sa_hbm, sb_hbm,
                  pidx_v, pidx2_v, rowsa_v, rowsb_v, sem):
    c = lax.axis_index("c")
    s = lax.axis_index("s")
    w = s * 2 + c
    base = w * (P1 // 32)
    pw = P1 // 32
    pltpu.sync_copy(perm_hbm.at[pl.ds(base, pw)], pidx_v)

    def adj(g, cc):
        pidx2_v[pl.ds(g * 16, 16)] = pidx_v[pl.ds(g * 16, 16)] + NP
        return cc

    lax.fori_loop(0, pw // 16, adj, 0)
    pltpu.async_copy(o2_hbm.at[pidx_v], rowsa_v, sem).wait()
    pltpu.async_copy(o2_hbm.at[pidx2_v], rowsb_v, sem).wait()
    pltpu.sync_copy(rowsa_v, sa_hbm.at[pl.ds(base, pw)])
    pltpu.sync_copy(rowsb_v, sb_hbm.at[pl.ds(base, pw)])


_gather_halves = pl.kernel(
    _gatherh_body,
    out_type=[
        jax.ShapeDtypeStruct((P1, 32), jnp.float32),
        jax.ShapeDtypeStruct((P1, 32), jnp.float32),
    ],
    mesh=_mesh,
    compiler_params=_sc_params,
    scratch_types=[
        pltpu.VMEM((P1 // 32,), jnp.int32),
        pltpu.VMEM((P1 // 32,), jnp.int32),
        pltpu.VMEM((P1 // 32, 32), jnp.float32),
        pltpu.VMEM((P1 // 32, 32), jnp.float32),
        pltpu.SemaphoreType.DMA,
    ],
)


def _seg2_body(src_hbm, dst_hbm, gcat_hbm, z_hbm, o_hbm,
               eidx_v, didx_v, rows_v, acc_sh, sem):
    c = lax.axis_index("c")
    s = lax.axis_index("s")
    pltpu.sync_copy(z_hbm.at[pl.ds(0, 3136)], acc_sh.at[pl.ds(s * 3136, 3136)])
    plsc.subcore_barrier()
    coff = c * NP

    def chunk(j, carry):
        off = s * ESUB + j * ECH1
        pltpu.sync_copy(src_hbm.at[pl.ds(off, ECH1)], eidx_v)
        pltpu.sync_copy(dst_hbm.at[pl.ds(off, ECH1)], didx_v)

        def adj(g, cc):
            eidx_v[pl.ds(g * 16, 16)] = eidx_v[pl.ds(g * 16, 16)] + coff
            return cc

        lax.fori_loop(0, ECH1 // 16, adj, 0)
        pltpu.async_copy(gcat_hbm.at[eidx_v], rows_v, sem).wait()
        pltpu.sync_copy(rows_v, acc_sh.at[didx_v], add=True)
        return carry

    lax.fori_loop(0, ESUB // ECH1, chunk, 0)
    plsc.subcore_barrier()
    r0 = s * 3136
    pltpu.sync_copy(acc_sh.at[pl.ds(r0, 3136)],
                    o_hbm.at[pl.ds(c * NP + r0, 3136)])


_seg2 = pl.kernel(
    _seg2_body,
    out_type=jax.ShapeDtypeStruct((2 * NP, 32), jnp.float32),
    mesh=_mesh,
    compiler_params=_sc_params,
    scratch_types=[
        pltpu.VMEM((ECH1,), jnp.int32),
        pltpu.VMEM((ECH1,), jnp.int32),
        pltpu.VMEM((ECH1, 32), jnp.float32),
        pltpu.VMEM_SHARED((NP, 32), jnp.float32),
        pltpu.SemaphoreType.DMA,
    ],
)


def _scale_reduce_body(kvalid, with_mm, nb, *refs):
    if with_mm:
        (xg_ref, at_ref, w_ref, ha_ref, hb_ref, xr_ref, a_ref,
         amax, asum) = refs
    else:
        (xg_ref, at_ref, xr_ref, a_ref, amax, asum) = refs
    i = pl.program_id(0)
    bm = xg_ref.shape[0]
    at = at_ref[...]
    x = xg_ref[...] * at
    rows = i * bm + lax.broadcasted_iota(jnp.int32, (bm, DIM), 0)
    valid = rows < kvalid
    xm = jnp.where(valid, x, -jnp.inf)
    xs_ = jnp.where(valid, x, 0.0)

    @pl.when(i == 0)
    def _():
        amax[...] = jnp.full((1, DIM), -jnp.inf, jnp.float32)
        asum[...] = jnp.zeros((1, DIM), jnp.float32)

    amax[...] = jnp.maximum(amax[...], jnp.max(xm, axis=0, keepdims=True))
    asum[...] = asum[...] + jnp.sum(xs_, axis=0, keepdims=True)
    a_ref[...] = 1.0 / (1.0 + jnp.exp(-at))
    if with_mm:
        h2 = jnp.dot(x, w_ref[...], preferred_element_type=jnp.float32)
        ha_ref[...] = h2[:, :32]
        hb_ref[...] = h2[:, 32:]

    @pl.when(i == nb - 1)
    def _():
        xr_ref[...] = jnp.concatenate(
            [amax[...], asum[...] * (1.0 / kvalid)], axis=1)


def _scale_reduce(xg, at2d, kvalid, w=None, bm=400):
    m = xg.shape[0]
    nb = m // bm
    with_mm = w is not None
    in_specs = [
        pl.BlockSpec((bm, DIM), lambda i: (i, 0)),
        pl.BlockSpec((bm, 1), lambda i: (i, 0)),
    ]
    args = [xg, at2d]
    out_specs = []
    out_shape = []
    if with_mm:
        in_specs.append(pl.BlockSpec((DIM, DIM), lambda i: (0, 0)))
        args.append(w)
        out_specs += [pl.BlockSpec((bm, 32), lambda i: (i, 0)),
                      pl.BlockSpec((bm, 32), lambda i: (i, 0))]
        out_shape += [jax.ShapeDtypeStruct((m, 32), jnp.float32),
                      jax.ShapeDtypeStruct((m, 32), jnp.float32)]
    out_specs += [pl.BlockSpec((1, 2 * DIM), lambda i: (0, 0)),
                  pl.BlockSpec((bm, 1), lambda i: (i, 0))]
    out_shape += [jax.ShapeDtypeStruct((1, 2 * DIM), jnp.float32),
                  jax.ShapeDtypeStruct((m, 1), jnp.float32)]
    return pl.pallas_call(
        functools.partial(_scale_reduce_body, kvalid, with_mm, nb),
        grid=(nb,),
        in_specs=in_specs,
        out_specs=out_specs,
        out_shape=out_shape,
        scratch_shapes=[
            pltpu.VMEM((1, DIM), jnp.float32),
            pltpu.VMEM((1, DIM), jnp.float32),
        ],
    )(*args)


def _logsm(z):
    m = jnp.max(z, axis=1, keepdims=True)
    e = jnp.exp(z - m)
    return z - m - jnp.log(jnp.sum(e, axis=1, keepdims=True))


def _head_body(x1r_ref, x2r_ref, wf1_ref, bf1_ref, g1_ref, be1_ref,
               wf2_ref, bf2_ref, g2_ref, be2_ref, wf3_ref, bf3_ref,
               ws_ref, bs_ref, xy_ref, xs_ref):
    inv_bn = 1.0 / math.sqrt(1.0 + 1e-5)
    xc = x1r_ref[...] + x2r_ref[...]
    y = jnp.maximum(
        jnp.dot(xc, wf1_ref[...], preferred_element_type=jnp.float32)
        + bf1_ref[...], 0.0)
    y = g1_ref[...] * y * inv_bn + be1_ref[...]
    y = jnp.maximum(
        jnp.dot(y, wf2_ref[...], preferred_element_type=jnp.float32)
        + bf2_ref[...], 0.0)
    y = g2_ref[...] * y * inv_bn + be2_ref[...]
    z = jnp.dot(y, wf3_ref[...], preferred_element_type=jnp.float32) \
        + bf3_ref[...]
    xy_ref[...] = _logsm(z)
    zs = jnp.dot(xc, ws_ref[...], preferred_element_type=jnp.float32) \
        + bs_ref[...]
    xs_ref[...] = _logsm(zs)


def _head(x1r, x2r, wf1, bf1, g1, be1, wf2, bf2, g2, be2, wf3p, bf3p,
          wsp, bsp):
    return pl.pallas_call(
        _head_body,
        out_shape=[
            jax.ShapeDtypeStruct((1, 128), jnp.float32),
            jax.ShapeDtypeStruct((1, 128), jnp.float32),
        ],
    )(x1r, x2r, wf1, bf1.reshape(1, -1), g1.reshape(1, -1),
      be1.reshape(1, -1), wf2, bf2.reshape(1, -1), g2.reshape(1, -1),
      be2.reshape(1, -1), wf3p, bf3p, wsp, bsp)


def kernel(x, edge_index, edge_attr, batch, W1, b1, p1, W2, b2, p2,
           Wf1, bf1, g1, be1, Wf2, bf2, g2, be2, Wf3, bf3, Ws, bs):
    src = edge_index[0]
    dst = edge_index[1]
    zrows = jnp.zeros((3136, 32), jnp.float32)

    h_lin = _mm(x, W1)
    hcat = jnp.concatenate([h_lin[:, :32], h_lin[:, 32:]], axis=0)
    o1 = _seg1(src, dst, hcat, zrows)
    h, score = _bias_score(o1[:NP], o1[NP:], b1, p1)

    attn1, perm1 = lax.top_k(score[:N, 0], K1)
    perm1p = jnp.concatenate(
        [perm1, jnp.zeros((P1 - K1,), jnp.int32)])
    attn1p = jnp.concatenate(
        [attn1, jnp.zeros((P1 - K1,), jnp.float32)])
    xg = _gather1(perm1p, h)
    h2aa, h2ab, x1r, a1col = _scale_reduce(
        xg, attn1p.reshape(P1, 1), K1, w=W2)

    h2cat = jnp.concatenate([h2aa, h2ab], axis=0)
    gcat = _rowscatter(perm1p, h2cat, zrows)
    o2 = _seg2(src, dst, gcat, zrows)
    s2a, s2b = _gather_halves(perm1p, o2)
    h2, score2 = _bias_score(s2a, s2b, b2, p2)

    attn2, perm2 = lax.top_k(score2[:K1, 0], K2)
    perm2p = jnp.concatenate(
        [perm2, jnp.zeros((P2 - K2,), jnp.int32)])
    attn2p = jnp.concatenate(
        [attn2, jnp.zeros((P2 - K2,), jnp.float32)])
    x2g = _gather2(perm2p, h2)
    x2r, a2col = _scale_reduce(x2g, attn2p.reshape(P2, 1), K2)

    wf3p = jnp.pad(Wf3, ((0, 0), (0, 126)))
    bf3p = jnp.pad(bf3, (0, 126), constant_values=-1e30).reshape(1, 128)
    wsp = jnp.pad(Ws, ((0, 0), (0, 124)))
    bsp = jnp.pad(bs, (0, 124), constant_values=-1e30).reshape(1, 128)
    xy_full, xs_full = _head(x1r, x2r, Wf1, bf1, g1, be1, Wf2, bf2,
                             g2, be2, wf3p, bf3p, wsp, bsp)
    xy = xy_full[:, :2]
    xs = xs_full[:, :4]
    a1 = a1col[:K1, 0].reshape(K2, 2)
    a2 = a2col[:K2, 0].reshape(K2, 1)
    return (xy, xs, a1, a2)
